# Initial kernel scaffold; baseline (speedup 1.0000x reference)
import jax
import jax.numpy as jnp
from jax.experimental import pallas as pl


def kernel(bits, norm_C_points):
    raise NotImplementedError("write your pallas kernel here")


# speedup vs baseline: 137.4241x; 137.4241x over previous
import functools

import jax
import jax.numpy as jnp
from jax import lax
from jax.experimental import pallas as pl
from jax.experimental.pallas import tpu as pltpu
from jax.experimental.pallas import tpu_sc as plsc

B = 4096
NBITS = 8192
SYMS = 2048
OUTW = 2 * SYMS
NC, NS, L = 2, 16, 16
NW = NC * NS
ROWS_PER_W = B // NW


def _dg(x, idx):
  return jnp.take_along_axis(x, idx, axis=0, mode="promise_in_bounds")


def _sc_mapper(bits_hbm, tab_hbm, out_hbm, bits_v, tab_v, out_v, sem):
  wid = lax.axis_index("s") * NC + lax.axis_index("c")
  base = wid * ROWS_PER_W

  iot = lax.iota(jnp.int32, L)
  pat = 1 << (3 - (iot & 3))
  perm1 = iot ^ 1
  perm2 = iot ^ 2
  permP = ((iot & 7) >> 1) << 2
  low8 = iot < 8
  comp = iot & 1

  def row_body(r, _):
    row = base + r
    pltpu.sync_copy(bits_hbm.at[row], bits_v)
    pltpu.sync_copy(tab_hbm.at[row], tab_v)

    def step(u, _):
      v1 = bits_v[pl.ds(u * 32, L)]
      v2 = bits_v[pl.ds(u * 32 + L, L)]
      w1 = v1 * pat
      w2 = v2 * pat
      r1 = w1 + _dg(w1, perm1)
      r1 = r1 + _dg(r1, perm2)
      r2 = w2 + _dg(w2, perm1)
      r2 = r2 + _dg(r2, perm2)
      idx16 = jnp.where(low8, _dg(r1, permP), _dg(r2, permP))
      addr = (idx16 << 1) | comp
      out_v[pl.ds(u * L, L)] = plsc.load_gather(tab_v, [addr])
      return 0

    lax.fori_loop(0, NBITS // 32, step, 0, unroll=4)
    pltpu.sync_copy(out_v, out_hbm.at[row])
    return 0

  lax.fori_loop(0, ROWS_PER_W, row_body, 0)


@jax.jit
def _run(bits, tab):
  f = pl.kernel(
      _sc_mapper,
      out_type=jax.ShapeDtypeStruct((B, OUTW), jnp.float32),
      mesh=plsc.VectorSubcoreMesh(core_axis_name="c", subcore_axis_name="s"),
      compiler_params=pltpu.CompilerParams(needs_layout_passes=False),
      scratch_types=[
          pltpu.VMEM((NBITS,), jnp.int32),
          pltpu.VMEM((2 * L,), jnp.float32),
          pltpu.VMEM((OUTW,), jnp.float32),
          pltpu.SemaphoreType.DMA,
      ],
  )
  return f(bits, tab)


def kernel(bits, norm_C_points):
  tab = norm_C_points.reshape(B, 2 * L)
  out = _run(bits, tab)
  return out.reshape(B, SYMS, 2)


# speedup vs baseline: 351.8832x; 2.5606x over previous
import jax
import jax.numpy as jnp
from jax import lax
from jax.experimental import pallas as pl
from jax.experimental.pallas import tpu as pltpu
from jax.experimental.pallas import tpu_sc as plsc

B = 4096
NBITS = 8192
SYMS = 2048
OUTW = 2 * SYMS
NC, NS, L = 2, 16, 16
NW = NC * NS
RPW = B // NW
STEPS = NBITS // 32


def _dg(x, idx):
  return jnp.take_along_axis(x, idx, axis=0, mode="promise_in_bounds")


def _sc_mapper(bits_hbm, tab_hbm, out_hbm,
               bits0, bits1, out0, out1, tab_v,
               in_sem0, in_sem1, out_sem0, out_sem1, tab_sem):
  wid = lax.axis_index("s") * NC + lax.axis_index("c")
  base = wid * RPW

  tab_cp = pltpu.async_copy(tab_hbm.at[pl.ds(base * 2 * L, RPW * 2 * L)],
                            tab_v, tab_sem)
  pltpu.async_copy(bits_hbm.at[base], bits0, in_sem0)
  tab_cp.wait()

  iot = lax.iota(jnp.int32, L)
  pat = 1 << (3 - (iot & 3))
  perm1 = iot ^ 1
  perm2 = iot ^ 2
  permP = ((iot & 7) >> 1) << 2
  low8 = iot < 8
  comp = iot & 1

  def compute_row(r, bits_b, out_b):
    tbase = r << 5

    @plsc.parallel_loop(0, STEPS, unroll=8)
    def step(u):
      v1 = bits_b[pl.ds(u * 32, L)]
      v2 = bits_b[pl.ds(u * 32 + L, L)]
      w1 = v1 * pat
      w2 = v2 * pat
      r1 = w1 + _dg(w1, perm1)
      r1 = r1 + _dg(r1, perm2)
      r2 = w2 + _dg(w2, perm1)
      r2 = r2 + _dg(r2, perm2)
      idx16 = jnp.where(low8, _dg(r1, permP), _dg(r2, permP))
      addr = tbase + (idx16 << 1) + comp
      out_b[pl.ds(u * L, L)] = plsc.load_gather(tab_v, [addr])

  bufs = ((bits0, out0, in_sem0, out_sem0),
          (bits1, out1, in_sem1, out_sem1))

  def pair_body(p, _):
    for b in range(2):
      bits_b, out_b, in_sem_b, out_sem_b = bufs[b]
      o_bits, _, o_in_sem, _ = bufs[1 - b]
      r = 2 * p + b
      row = base + r

      @pl.when(r + 1 < RPW)
      def _prefetch():
        pltpu.async_copy(bits_hbm.at[row + 1], o_bits, o_in_sem)

      pltpu.make_async_copy(bits_hbm.at[row], bits_b, in_sem_b).wait()

      @pl.when(r >= 2)
      def _drain():
        pltpu.make_async_copy(out_b, out_hbm.at[row - 2], out_sem_b).wait()

      compute_row(r, bits_b, out_b)
      pltpu.async_copy(out_b, out_hbm.at[row], out_sem_b)
    return 0

  lax.fori_loop(0, RPW // 2, pair_body, 0)
  pltpu.make_async_copy(out0, out_hbm.at[base + RPW - 2], out_sem0).wait()
  pltpu.make_async_copy(out1, out_hbm.at[base + RPW - 1], out_sem1).wait()


@jax.jit
def _run(bits, tab):
  f = pl.kernel(
      _sc_mapper,
      out_type=jax.ShapeDtypeStruct((B, OUTW), jnp.float32),
      mesh=plsc.VectorSubcoreMesh(core_axis_name="c", subcore_axis_name="s"),
      compiler_params=pltpu.CompilerParams(needs_layout_passes=False),
      scratch_types=[
          pltpu.VMEM((NBITS,), jnp.int32),
          pltpu.VMEM((NBITS,), jnp.int32),
          pltpu.VMEM((OUTW,), jnp.float32),
          pltpu.VMEM((OUTW,), jnp.float32),
          pltpu.VMEM((RPW * 2 * L,), jnp.float32),
          pltpu.SemaphoreType.DMA,
          pltpu.SemaphoreType.DMA,
          pltpu.SemaphoreType.DMA,
          pltpu.SemaphoreType.DMA,
          pltpu.SemaphoreType.DMA,
      ],
  )
  return f(bits, tab)


def kernel(bits, norm_C_points):
  tab = norm_C_points.reshape(B * 2 * L)
  out = _run(bits, tab)
  return out.reshape(B, SYMS, 2)


# speedup vs baseline: 355.3147x; 1.0098x over previous
import jax
import jax.numpy as jnp
from jax import lax
from jax.experimental import pallas as pl
from jax.experimental.pallas import tpu as pltpu
from jax.experimental.pallas import tpu_sc as plsc

B = 4096
NBITS = 8192
SYMS = 2048
OUTW = 2 * SYMS
NC, NS, L = 2, 16, 16
NW = NC * NS
RPW = B // NW
STEPS = NBITS // 32


def _dg(x, idx):
  return jnp.take_along_axis(x, idx, axis=0, mode="promise_in_bounds")


def _sc_mapper(bits_hbm, tab_hbm, out_hbm,
               bits0, bits1, out0, out1, tab_v,
               in_sem0, in_sem1, out_sem0, out_sem1, tab_sem):
  wid = lax.axis_index("s") * NC + lax.axis_index("c")
  base = wid * RPW

  tab_cp = pltpu.async_copy(tab_hbm.at[pl.ds(base * 2 * L, RPW * 2 * L)],
                            tab_v, tab_sem)
  pltpu.async_copy(bits_hbm.at[base], bits0, in_sem0)
  tab_cp.wait()

  iot = lax.iota(jnp.int32, L)
  pat = 1 << (3 - (iot & 3))
  perm1 = iot ^ 1
  perm2 = iot ^ 2
  permP = ((iot & 7) >> 1) << 2
  low8 = iot < 8
  comp = iot & 1

  def compute_row(r, bits_b, out_b):
    tbase = r << 5

    @plsc.parallel_loop(0, STEPS, unroll=16)
    def step(u):
      v1 = bits_b[pl.ds(u * 32, L)]
      v2 = bits_b[pl.ds(u * 32 + L, L)]
      w1 = v1 * pat
      w2 = v2 * pat
      r1 = w1 + _dg(w1, perm1)
      r1 = r1 + _dg(r1, perm2)
      r2 = w2 + _dg(w2, perm1)
      r2 = r2 + _dg(r2, perm2)
      idx16 = jnp.where(low8, _dg(r1, permP), _dg(r2, permP))
      addr = tbase + (idx16 << 1) + comp
      out_b[pl.ds(u * L, L)] = plsc.load_gather(tab_v, [addr])

  bufs = ((bits0, out0, in_sem0, out_sem0),
          (bits1, out1, in_sem1, out_sem1))

  def pair_body(p, _):
    for b in range(2):
      bits_b, out_b, in_sem_b, out_sem_b = bufs[b]
      o_bits, _, o_in_sem, _ = bufs[1 - b]
      r = 2 * p + b
      row = base + r

      @pl.when(r + 1 < RPW)
      def _prefetch():
        pltpu.async_copy(bits_hbm.at[row + 1], o_bits, o_in_sem)

      pltpu.make_async_copy(bits_hbm.at[row], bits_b, in_sem_b).wait()

      @pl.when(r >= 2)
      def _drain():
        pltpu.make_async_copy(out_b, out_hbm.at[row - 2], out_sem_b).wait()

      compute_row(r, bits_b, out_b)
      pltpu.async_copy(out_b, out_hbm.at[row], out_sem_b)
    return 0

  lax.fori_loop(0, RPW // 2, pair_body, 0)
  pltpu.make_async_copy(out0, out_hbm.at[base + RPW - 2], out_sem0).wait()
  pltpu.make_async_copy(out1, out_hbm.at[base + RPW - 1], out_sem1).wait()


@jax.jit
def _run(bits, tab):
  f = pl.kernel(
      _sc_mapper,
      out_type=jax.ShapeDtypeStruct((B, OUTW), jnp.float32),
      mesh=plsc.VectorSubcoreMesh(core_axis_name="c", subcore_axis_name="s"),
      compiler_params=pltpu.CompilerParams(needs_layout_passes=False),
      scratch_types=[
          pltpu.VMEM((NBITS,), jnp.int32),
          pltpu.VMEM((NBITS,), jnp.int32),
          pltpu.VMEM((OUTW,), jnp.float32),
          pltpu.VMEM((OUTW,), jnp.float32),
          pltpu.VMEM((RPW * 2 * L,), jnp.float32),
          pltpu.SemaphoreType.DMA,
          pltpu.SemaphoreType.DMA,
          pltpu.SemaphoreType.DMA,
          pltpu.SemaphoreType.DMA,
          pltpu.SemaphoreType.DMA,
      ],
  )
  return f(bits, tab)


def kernel(bits, norm_C_points):
  tab = norm_C_points.reshape(B * 2 * L)
  out = _run(bits, tab)
  return out.reshape(B, SYMS, 2)


# speedup vs baseline: 902.8183x; 2.5409x over previous
import jax
import jax.numpy as jnp
from jax import lax
from jax.experimental import pallas as pl
from jax.experimental.pallas import tpu as pltpu
from jax.experimental.pallas import tpu_sc as plsc

B = 4096
NBITS = 8192
SYMS = 2048
OUTW = 2 * SYMS
NC, NS, L = 2, 16, 16
NW = NC * NS
RPW = B // NW
STEPS = NBITS // 64


def _dg(x, idx):
  return jnp.take_along_axis(x, idx, axis=0, mode="promise_in_bounds")


def _sc_mapper(bits_hbm, tab_hbm, out_hbm,
               bits0, bits1, out0, out1, tab_raw, tab_v,
               in_sem0, in_sem1, out_sem0, out_sem1, tab_sem):
  wid = lax.axis_index("s") * NC + lax.axis_index("c")
  base = wid * RPW

  for p in range(16):
    pltpu.async_copy(tab_hbm.at[pl.ds(p * 8192 + wid * 256, 256)],
                     tab_raw.at[pl.ds(p * 256, 256)], tab_sem)
  pltpu.async_copy(bits_hbm.at[base], bits0, in_sem0)
  for p in range(16):
    pltpu.make_async_copy(tab_hbm.at[pl.ds(p * 8192 + wid * 256, 256)],
                          tab_raw.at[pl.ds(p * 256, 256)], tab_sem).wait()

  iot = lax.iota(jnp.int32, L)
  pat = 1 << (3 - (iot & 3))
  perm1 = iot ^ 1
  perm2 = iot ^ 2
  permP = (iot & 3) << 2
  q1m = iot < 4
  q2m = iot < 8
  q3m = iot < 12

  @plsc.parallel_loop(0, 2 * RPW, unroll=8)
  def repack(i):
    r = i >> 1
    c = i & 1
    tab_v[pl.ds(i * 16, L)] = plsc.load_gather(
        tab_raw, [iot * 256 + c * 128 + r])

  def compute_row(r, bits_b, out_b):
    rbase = r << 5

    @plsc.parallel_loop(0, STEPS, unroll=8)
    def step(u):
      v1 = bits_b[pl.ds(u * 64, L)]
      v2 = bits_b[pl.ds(u * 64 + 16, L)]
      v3 = bits_b[pl.ds(u * 64 + 32, L)]
      v4 = bits_b[pl.ds(u * 64 + 48, L)]
      w1 = v1 * pat
      w2 = v2 * pat
      w3 = v3 * pat
      w4 = v4 * pat
      r1 = w1 + _dg(w1, perm1)
      r1 = r1 + _dg(r1, perm2)
      r2 = w2 + _dg(w2, perm1)
      r2 = r2 + _dg(r2, perm2)
      r3 = w3 + _dg(w3, perm1)
      r3 = r3 + _dg(r3, perm2)
      r4 = w4 + _dg(w4, perm1)
      r4 = r4 + _dg(r4, perm2)
      idx16 = jnp.where(
          q1m, _dg(r1, permP),
          jnp.where(q2m, _dg(r2, permP),
                    jnp.where(q3m, _dg(r3, permP), _dg(r4, permP))))
      addr = rbase + idx16
      off = u * 16 + (u >> 3) * 128
      out_b[pl.ds(off, L)] = plsc.load_gather(tab_v, [addr])
      out_b[pl.ds(off + 128, L)] = plsc.load_gather(tab_v, [addr + 16])

  bufs = ((bits0, out0, in_sem0, out_sem0),
          (bits1, out1, in_sem1, out_sem1))

  def pair_body(p, _):
    for b in range(2):
      bits_b, out_b, in_sem_b, out_sem_b = bufs[b]
      o_bits, _, o_in_sem, _ = bufs[1 - b]
      r = 2 * p + b
      row = base + r

      @pl.when(r + 1 < RPW)
      def _prefetch():
        pltpu.async_copy(bits_hbm.at[row + 1], o_bits, o_in_sem)

      pltpu.make_async_copy(bits_hbm.at[row], bits_b, in_sem_b).wait()

      @pl.when(r >= 2)
      def _drain():
        pltpu.make_async_copy(
            out_b, out_hbm.at[pl.ds((row - 2) * OUTW, OUTW)], out_sem_b).wait()

      compute_row(r, bits_b, out_b)
      pltpu.async_copy(out_b, out_hbm.at[pl.ds(row * OUTW, OUTW)], out_sem_b)
    return 0

  lax.fori_loop(0, RPW // 2, pair_body, 0)
  last0 = base + RPW - 2
  last1 = base + RPW - 1
  pltpu.make_async_copy(out0, out_hbm.at[pl.ds(last0 * OUTW, OUTW)],
                        out_sem0).wait()
  pltpu.make_async_copy(out1, out_hbm.at[pl.ds(last1 * OUTW, OUTW)],
                        out_sem1).wait()


@jax.jit
def _run(bits, tab):
  f = pl.kernel(
      _sc_mapper,
      out_type=jax.ShapeDtypeStruct((B * OUTW,), jnp.float32),
      mesh=plsc.VectorSubcoreMesh(core_axis_name="c", subcore_axis_name="s"),
      compiler_params=pltpu.CompilerParams(needs_layout_passes=False),
      scratch_types=[
          pltpu.VMEM((NBITS,), jnp.int32),
          pltpu.VMEM((NBITS,), jnp.int32),
          pltpu.VMEM((OUTW,), jnp.float32),
          pltpu.VMEM((OUTW,), jnp.float32),
          pltpu.VMEM((16 * 256,), jnp.float32),
          pltpu.VMEM((RPW * 2 * L,), jnp.float32),
          pltpu.SemaphoreType.DMA,
          pltpu.SemaphoreType.DMA,
          pltpu.SemaphoreType.DMA,
          pltpu.SemaphoreType.DMA,
          pltpu.SemaphoreType.DMA,
      ],
  )
  return f(bits, tab)


def kernel(bits, norm_C_points):
  tab = (norm_C_points.reshape(32, 128, 16, 2)
         .transpose(2, 0, 3, 1)
         .reshape(B * 2 * L))
  flat = _run(bits, tab)
  return (flat.reshape(B, 16, 2, 128)
          .transpose(0, 1, 3, 2)
          .reshape(B, SYMS, 2))
